# Initial kernel scaffold; baseline (speedup 1.0000x reference)
import jax
import jax.numpy as jnp
from jax.experimental import pallas as pl


def kernel(x, edge_index, edge_attr, params):
    raise NotImplementedError("write your pallas kernel here")


# speedup vs baseline: 1.3094x; 1.3094x over previous
import functools

import jax
import jax.numpy as jnp
from jax import lax
from jax.experimental import pallas as pl
from jax.experimental.pallas import tpu as pltpu
from jax.experimental.pallas import tpu_sc as plsc

K = 15
DIM = 3
N_NODES = 10000
N_EDGES = 320000
NC, NS = 2, 16
NW = NC * NS
CHUNK = 32
EB = 3200
ROWS_PER_SUB = 624
ZROWS = 16


def _cop(co):
    return ((co + 15) // 16) * 16


def _gauss_body(ea_ref, mu_ref, inv_ref, out_ref):
    acc = jnp.zeros((EB, 16), jnp.float32)
    for d in range(DIM):
        col = ea_ref[:, d:d + 1]
        m = mu_ref[0, d:d + 1, :]
        iv = inv_ref[0, d:d + 1, :]
        t = col - m
        acc = acc + t * t * iv
    out_ref[0] = jnp.exp(-acc)


def _compute_gauss(ea_p, mu_b, inv_b, nl):
    return pl.pallas_call(
        _gauss_body,
        grid=(nl, N_EDGES // EB),
        in_specs=[
            pl.BlockSpec((EB, 8), lambda l, i: (i, 0)),
            pl.BlockSpec((1, 8, 16), lambda l, i: (l, 0, 0)),
            pl.BlockSpec((1, 8, 16), lambda l, i: (l, 0, 0)),
        ],
        out_specs=pl.BlockSpec((1, EB, 16), lambda l, i: (l, i, 0)),
        out_shape=jax.ShapeDtypeStruct((nl, N_EDGES, 16), jnp.float32),
    )(ea_p, mu_b, inv_b)


def _dense_body(h_ref, w_ref, r_ref, b_ref, xh_ref, hr_ref):
    h = h_ref[...]
    xh_ref[...] = jnp.dot(h, w_ref[...], preferred_element_type=jnp.float32)
    hr_ref[...] = (jnp.dot(h, r_ref[...], preferred_element_type=jnp.float32)
                   + b_ref[...])


def _dense(h, w_pad, root_pad, bias_pad, cop):
    n, cip = h.shape
    row = K * cop
    bt = 400
    return pl.pallas_call(
        _dense_body,
        grid=(n // bt,),
        in_specs=[
            pl.BlockSpec((bt, cip), lambda i: (i, 0)),
            pl.BlockSpec((cip, row), lambda i: (0, 0)),
            pl.BlockSpec((cip, cop), lambda i: (0, 0)),
            pl.BlockSpec((1, cop), lambda i: (0, 0)),
        ],
        out_specs=[
            pl.BlockSpec((bt, row), lambda i: (i, 0)),
            pl.BlockSpec((bt, cop), lambda i: (i, 0)),
        ],
        out_shape=[
            jax.ShapeDtypeStruct((n, row), jnp.float32),
            jax.ShapeDtypeStruct((n, cop), jnp.float32),
        ],
    )(h, w_pad, root_pad, bias_pad)


@functools.lru_cache(maxsize=None)
def _make_sc_kernel(cop):
    row = K * cop
    nv = cop // 16
    nchunks_tot = N_EDGES // CHUNK

    def body(xh, srcr, dstr, gaussr, out, sidx, didx, gbuf, rows, msg, zbuf,
             aggsh, sem):
        c = lax.axis_index("c")
        s = lax.axis_index("s")
        wid = s * NC + c

        def zrow(i, _):
            for v in range(nv):
                zbuf[i, pl.ds(16 * v, 16)] = jnp.zeros((16,), jnp.float32)
            return 0
        lax.fori_loop(0, ZROWS, zrow, 0)
        rstart = s * ROWS_PER_SUB

        def zcp(j, _):
            pltpu.sync_copy(zbuf, aggsh.at[pl.ds(rstart + j * ZROWS, ZROWS)])
            return 0
        nz = ROWS_PER_SUB // ZROWS + jnp.where(s == NS - 1, 1, 0)
        lax.fori_loop(0, nz, zcp, 0)
        plsc.subcore_barrier()

        nch = (nchunks_tot - wid + NW - 1) // NW

        def chunk(i, _):
            base = (wid + NW * i) * CHUNK
            pltpu.sync_copy(srcr.at[pl.ds(base, CHUNK)], sidx)
            pltpu.sync_copy(dstr.at[pl.ds(base, CHUNK)], didx)
            pltpu.sync_copy(gaussr.at[pl.ds(base, CHUNK)], gbuf)
            pltpu.async_copy(xh.at[sidx], rows, sem).wait()

            def edge(e, _):
                gv = gbuf[e, pl.ds(0, 16)]
                g = [gv[k] for k in range(K)]
                for v in range(nv):
                    acc = g[0] * rows[e, pl.ds(16 * v, 16)]
                    for k in range(1, K):
                        acc = acc + g[k] * rows[e, pl.ds(k * cop + 16 * v, 16)]
                    msg[e, pl.ds(16 * v, 16)] = acc
                return 0
            lax.fori_loop(0, CHUNK, edge, 0)
            pltpu.sync_copy(msg, aggsh.at[didx], add=True)
            return 0
        lax.fori_loop(0, nch, chunk, 0)
        plsc.subcore_barrier()

        @pl.when(s < NS - 1)
        def _():
            pltpu.sync_copy(aggsh.at[pl.ds(rstart, ROWS_PER_SUB)],
                            out.at[c, pl.ds(rstart, ROWS_PER_SUB)])

        @pl.when(s == NS - 1)
        def _():
            last = (NS - 1) * ROWS_PER_SUB
            pltpu.sync_copy(aggsh.at[pl.ds(last, N_NODES - last)],
                            out.at[c, pl.ds(last, N_NODES - last)])

    mesh = plsc.VectorSubcoreMesh(core_axis_name="c", subcore_axis_name="s")
    return pl.kernel(
        body,
        out_type=jax.ShapeDtypeStruct((NC, N_NODES, cop), jnp.float32),
        mesh=mesh,
        compiler_params=pltpu.CompilerParams(use_tc_tiling_on_sc=False),
        scratch_types=[
            pltpu.VMEM((CHUNK,), jnp.int32),
            pltpu.VMEM((CHUNK,), jnp.int32),
            pltpu.VMEM((CHUNK, 16), jnp.float32),
            pltpu.VMEM((CHUNK, row), jnp.float32),
            pltpu.VMEM((CHUNK, cop), jnp.float32),
            pltpu.VMEM((ZROWS, cop), jnp.float32),
            pltpu.VMEM_SHARED((N_NODES, cop), jnp.float32),
            pltpu.SemaphoreType.DMA,
        ],
    )


def _post_body_bn(agg_ref, hr_ref, g_ref, b_ref, out_ref):
    sm = agg_ref[0] + agg_ref[1] + hr_ref[...]
    e = jnp.where(sm > 0, sm, jnp.exp(sm) - 1.0)
    m = jnp.mean(e, axis=0, keepdims=True)
    d = e - m
    v = jnp.mean(d * d, axis=0, keepdims=True)
    out_ref[...] = d * lax.rsqrt(v + 1e-5) * g_ref[...] + b_ref[...]


def _post_body_last(agg_ref, hr_ref, out_ref):
    out_ref[...] = agg_ref[0] + agg_ref[1] + hr_ref[...]


def kernel(x, edge_index, edge_attr, params):
    src = edge_index[0].astype(jnp.int32)
    dst = edge_index[1].astype(jnp.int32)
    nl = len(params)

    ea_p = jnp.zeros((N_EDGES, 8), jnp.float32).at[:, :DIM].set(edge_attr)
    mu_b = jnp.zeros((nl, 8, 16), jnp.float32)
    inv_b = jnp.zeros((nl, 8, 16), jnp.float32)
    for l, p in enumerate(params):
        mu_b = mu_b.at[l, :DIM, :K].set(p["mu"].T)
        inv_b = inv_b.at[l, :DIM, :K].set((0.5 / (p["sigma"] ** 2 + 1e-12)).T)
    gauss_all = _compute_gauss(ea_p, mu_b, inv_b, nl)

    h = x
    for l, p in enumerate(params):
        ci, co = p["root"].shape
        cip = h.shape[1]
        cop = _cop(co)
        w = p["W"].reshape(ci, K, co)
        w_pad = (jnp.zeros((cip, K, cop), jnp.float32)
                 .at[:ci, :, :co].set(w).reshape(cip, K * cop))
        root_pad = jnp.zeros((cip, cop), jnp.float32).at[:ci, :co].set(p["root"])
        bias_pad = jnp.zeros((1, cop), jnp.float32).at[0, :co].set(p["bias"])

        xh, hroot = _dense(h, w_pad, root_pad, bias_pad, cop)
        agg2 = _make_sc_kernel(cop)(xh, src, dst, gauss_all[l])

        if l < nl - 1:
            gamma_pad = jnp.zeros((1, cop), jnp.float32).at[0, :co].set(p["gamma"])
            beta_pad = jnp.zeros((1, cop), jnp.float32).at[0, :co].set(p["beta"])
            h = pl.pallas_call(
                _post_body_bn,
                out_shape=jax.ShapeDtypeStruct((N_NODES, cop), jnp.float32),
            )(agg2, hroot, gamma_pad, beta_pad)
        else:
            h = pl.pallas_call(
                _post_body_last,
                out_shape=jax.ShapeDtypeStruct((N_NODES, cop), jnp.float32),
            )(agg2, hroot)

    return h[:, :params[-1]["root"].shape[1]]


# speedup vs baseline: 1.7934x; 1.3697x over previous
import functools

import jax
import jax.numpy as jnp
from jax import lax
from jax.experimental import pallas as pl
from jax.experimental.pallas import tpu as pltpu
from jax.experimental.pallas import tpu_sc as plsc

K = 15
DIM = 3
N_NODES = 10000
N_EDGES = 320000
NC, NS = 2, 16
NW = NC * NS
CHUNK = 16
SBC = 25
SUPER = CHUNK * SBC
NSUPER = N_EDGES // NW // SUPER
EB = 3200
ROWS_PER_SUB = 624
ZROWS = 16


def _cop(co):
    return ((co + 15) // 16) * 16


def _gauss_body(ea_ref, mu_ref, inv_ref, out_ref):
    acc = jnp.zeros((EB, 16), jnp.float32)
    for d in range(DIM):
        col = ea_ref[:, d:d + 1]
        m = mu_ref[0, d:d + 1, :]
        iv = inv_ref[0, d:d + 1, :]
        t = col - m
        acc = acc + t * t * iv
    out_ref[0] = jnp.exp(-acc)


def _compute_gauss(ea_p, mu_b, inv_b, nl):
    return pl.pallas_call(
        _gauss_body,
        grid=(nl, N_EDGES // EB),
        in_specs=[
            pl.BlockSpec((EB, 8), lambda l, i: (i, 0)),
            pl.BlockSpec((1, 8, 16), lambda l, i: (l, 0, 0)),
            pl.BlockSpec((1, 8, 16), lambda l, i: (l, 0, 0)),
        ],
        out_specs=pl.BlockSpec((1, EB, 16), lambda l, i: (l, i, 0)),
        out_shape=jax.ShapeDtypeStruct((nl, N_EDGES, 16), jnp.float32),
    )(ea_p, mu_b, inv_b)


def _dense_body(h_ref, w_ref, r_ref, b_ref, xh_ref, hr_ref):
    h = h_ref[...]
    xh_ref[...] = jnp.dot(h, w_ref[...], preferred_element_type=jnp.float32)
    hr_ref[...] = (jnp.dot(h, r_ref[...], preferred_element_type=jnp.float32)
                   + b_ref[...])


def _dense(h, w_pad, root_pad, bias_pad, cop):
    n, cip = h.shape
    row = K * cop
    bt = 400
    return pl.pallas_call(
        _dense_body,
        grid=(n // bt,),
        in_specs=[
            pl.BlockSpec((bt, cip), lambda i: (i, 0)),
            pl.BlockSpec((cip, row), lambda i: (0, 0)),
            pl.BlockSpec((cip, cop), lambda i: (0, 0)),
            pl.BlockSpec((1, cop), lambda i: (0, 0)),
        ],
        out_specs=[
            pl.BlockSpec((bt, row), lambda i: (i, 0)),
            pl.BlockSpec((bt, cop), lambda i: (i, 0)),
        ],
        out_shape=[
            jax.ShapeDtypeStruct((n, row), jnp.float32),
            jax.ShapeDtypeStruct((n, cop), jnp.float32),
        ],
    )(h, w_pad, root_pad, bias_pad)


@functools.lru_cache(maxsize=None)
def _make_sc_kernel(cop):
    row = K * cop
    nv = cop // 16
    nbuf = 1 if cop > 96 else 2

    def impl(xh, srcr, dstr, gaussr, out, sidx, didx, gbuf, rows0, rows1, msg,
             zbuf, aggsh, sem0, sem1):
        c = lax.axis_index("c")
        s = lax.axis_index("s")
        wid = s * NC + c

        def zrow(i, _):
            for v in range(nv):
                zbuf[i, pl.ds(16 * v, 16)] = jnp.zeros((16,), jnp.float32)
            return 0
        lax.fori_loop(0, ZROWS, zrow, 0)
        rstart = s * ROWS_PER_SUB

        def zcp(j, _):
            pltpu.sync_copy(zbuf, aggsh.at[pl.ds(rstart + j * ZROWS, ZROWS)])
            return 0
        nz = ROWS_PER_SUB // ZROWS + jnp.where(s == NS - 1, 1, 0)
        lax.fori_loop(0, nz, zcp, 0)
        plsc.subcore_barrier()

        ebase_w = wid * (N_EDGES // NW)

        def superchunk(sb, _):
            sbase = ebase_w + sb * SUPER
            pltpu.sync_copy(srcr.at[pl.ds(sbase, SUPER)], sidx)
            pltpu.sync_copy(dstr.at[pl.ds(sbase, SUPER)], didx)
            pltpu.sync_copy(gaussr.at[pl.ds(sbase, SUPER)], gbuf)

            rbufs = (rows0, rows1)
            sems = (sem0, sem1)

            def start(ch, parity):
                svec = sidx[pl.ds(ch * CHUNK, CHUNK)]
                pltpu.async_copy(xh.at[svec], rbufs[parity], sems[parity])

            def wait(ch, parity):
                svec = sidx[pl.ds(ch * CHUNK, CHUNK)]
                pltpu.make_async_copy(xh.at[svec], rbufs[parity],
                                      sems[parity]).wait()

            def process(ch, parity):
                rbuf = rbufs[parity]

                def edge(e, _):
                    gv = gbuf[ch * CHUNK + e, pl.ds(0, 16)]
                    g = [gv[k] for k in range(K)]
                    for v in range(nv):
                        acc = g[0] * rbuf[e, pl.ds(16 * v, 16)]
                        for k in range(1, K):
                            acc = acc + g[k] * rbuf[e, pl.ds(k * cop + 16 * v, 16)]
                        msg[e, pl.ds(16 * v, 16)] = acc
                    return 0
                lax.fori_loop(0, CHUNK, edge, 0)
                dvec = didx[pl.ds(ch * CHUNK, CHUNK)]
                pltpu.sync_copy(msg, aggsh.at[dvec], add=True)

            if nbuf == 1:
                def chunk_b(ch, _):
                    start(ch, 0)
                    wait(ch, 0)
                    process(ch, 0)
                    return 0
                lax.fori_loop(0, SBC, chunk_b, 0)
            else:
                start(0, 0)
                start(1, 1)

                def pair(j, _):
                    c0 = 2 * j
                    wait(c0, 0)
                    process(c0, 0)

                    @pl.when(c0 + 2 < SBC)
                    def _():
                        start(c0 + 2, 0)

                    @pl.when(c0 + 1 < SBC)
                    def _():
                        wait(c0 + 1, 1)
                        process(c0 + 1, 1)

                        @pl.when(c0 + 3 < SBC)
                        def _():
                            start(c0 + 3, 1)
                    return 0
                lax.fori_loop(0, (SBC + 1) // 2, pair, 0)
            return 0
        lax.fori_loop(0, NSUPER, superchunk, 0)
        plsc.subcore_barrier()

        @pl.when(s < NS - 1)
        def _():
            pltpu.sync_copy(aggsh.at[pl.ds(rstart, ROWS_PER_SUB)],
                            out.at[c, pl.ds(rstart, ROWS_PER_SUB)])

        @pl.when(s == NS - 1)
        def _():
            last = (NS - 1) * ROWS_PER_SUB
            pltpu.sync_copy(aggsh.at[pl.ds(last, N_NODES - last)],
                            out.at[c, pl.ds(last, N_NODES - last)])

    if nbuf == 2:
        def body(xh, srcr, dstr, gaussr, out, sidx, didx, gbuf, rows0, rows1,
                 msg, zbuf, aggsh, sem0, sem1):
            impl(xh, srcr, dstr, gaussr, out, sidx, didx, gbuf, rows0, rows1,
                 msg, zbuf, aggsh, sem0, sem1)
        scratch = [
            pltpu.VMEM((SUPER,), jnp.int32),
            pltpu.VMEM((SUPER,), jnp.int32),
            pltpu.VMEM((SUPER, 16), jnp.float32),
            pltpu.VMEM((CHUNK, row), jnp.float32),
            pltpu.VMEM((CHUNK, row), jnp.float32),
            pltpu.VMEM((CHUNK, cop), jnp.float32),
            pltpu.VMEM((ZROWS, cop), jnp.float32),
            pltpu.VMEM_SHARED((N_NODES, cop), jnp.float32),
            pltpu.SemaphoreType.DMA,
            pltpu.SemaphoreType.DMA,
        ]
    else:
        def body(xh, srcr, dstr, gaussr, out, sidx, didx, gbuf, rows0,
                 msg, zbuf, aggsh, sem0):
            impl(xh, srcr, dstr, gaussr, out, sidx, didx, gbuf, rows0, rows0,
                 msg, zbuf, aggsh, sem0, sem0)
        scratch = [
            pltpu.VMEM((SUPER,), jnp.int32),
            pltpu.VMEM((SUPER,), jnp.int32),
            pltpu.VMEM((SUPER, 16), jnp.float32),
            pltpu.VMEM((CHUNK, row), jnp.float32),
            pltpu.VMEM((CHUNK, cop), jnp.float32),
            pltpu.VMEM((ZROWS, cop), jnp.float32),
            pltpu.VMEM_SHARED((N_NODES, cop), jnp.float32),
            pltpu.SemaphoreType.DMA,
        ]

    mesh = plsc.VectorSubcoreMesh(core_axis_name="c", subcore_axis_name="s")
    return pl.kernel(
        body,
        out_type=jax.ShapeDtypeStruct((NC, N_NODES, cop), jnp.float32),
        mesh=mesh,
        compiler_params=pltpu.CompilerParams(use_tc_tiling_on_sc=False),
        scratch_types=scratch,
    )


def _post_body_bn(agg_ref, hr_ref, g_ref, b_ref, out_ref):
    sm = agg_ref[0] + agg_ref[1] + hr_ref[...]
    e = jnp.where(sm > 0, sm, jnp.exp(sm) - 1.0)
    m = jnp.mean(e, axis=0, keepdims=True)
    d = e - m
    v = jnp.mean(d * d, axis=0, keepdims=True)
    out_ref[...] = d * lax.rsqrt(v + 1e-5) * g_ref[...] + b_ref[...]


def _post_body_last(agg_ref, hr_ref, out_ref):
    out_ref[...] = agg_ref[0] + agg_ref[1] + hr_ref[...]


def kernel(x, edge_index, edge_attr, params):
    src = edge_index[0].astype(jnp.int32)
    dst = edge_index[1].astype(jnp.int32)
    nl = len(params)

    ea_p = jnp.zeros((N_EDGES, 8), jnp.float32).at[:, :DIM].set(edge_attr)
    mu_b = jnp.zeros((nl, 8, 16), jnp.float32)
    inv_b = jnp.zeros((nl, 8, 16), jnp.float32)
    for l, p in enumerate(params):
        mu_b = mu_b.at[l, :DIM, :K].set(p["mu"].T)
        inv_b = inv_b.at[l, :DIM, :K].set((0.5 / (p["sigma"] ** 2 + 1e-12)).T)
    gauss_all = _compute_gauss(ea_p, mu_b, inv_b, nl)

    h = x
    for l, p in enumerate(params):
        ci, co = p["root"].shape
        cip = h.shape[1]
        cop = _cop(co)
        w = p["W"].reshape(ci, K, co)
        w_pad = (jnp.zeros((cip, K, cop), jnp.float32)
                 .at[:ci, :, :co].set(w).reshape(cip, K * cop))
        root_pad = jnp.zeros((cip, cop), jnp.float32).at[:ci, :co].set(p["root"])
        bias_pad = jnp.zeros((1, cop), jnp.float32).at[0, :co].set(p["bias"])

        xh, hroot = _dense(h, w_pad, root_pad, bias_pad, cop)
        agg2 = _make_sc_kernel(cop)(xh, src, dst, gauss_all[l])

        if l < nl - 1:
            gamma_pad = jnp.zeros((1, cop), jnp.float32).at[0, :co].set(p["gamma"])
            beta_pad = jnp.zeros((1, cop), jnp.float32).at[0, :co].set(p["beta"])
            h = pl.pallas_call(
                _post_body_bn,
                out_shape=jax.ShapeDtypeStruct((N_NODES, cop), jnp.float32),
            )(agg2, hroot, gamma_pad, beta_pad)
        else:
            h = pl.pallas_call(
                _post_body_last,
                out_shape=jax.ShapeDtypeStruct((N_NODES, cop), jnp.float32),
            )(agg2, hroot)

    return h[:, :params[-1]["root"].shape[1]]


# speedup vs baseline: 1.9002x; 1.0595x over previous
import functools

import jax
import jax.numpy as jnp
from jax import lax
from jax.experimental import pallas as pl
from jax.experimental.pallas import tpu as pltpu
from jax.experimental.pallas import tpu_sc as plsc

K = 15
DIM = 3
N_NODES = 10000
N_EDGES = 320000
NC, NS = 2, 16
NW = NC * NS
CHUNK = 16
EB = 3200
ROWS_PER_SUB = 624
ZROWS = 16


def _cop(co):
    return ((co + 15) // 16) * 16


def _gauss_body(ea_ref, mu_ref, inv_ref, out_ref):
    acc = jnp.zeros((EB, 16), jnp.float32)
    for d in range(DIM):
        col = ea_ref[:, d:d + 1]
        m = mu_ref[0, d:d + 1, :]
        iv = inv_ref[0, d:d + 1, :]
        t = col - m
        acc = acc + t * t * iv
    out_ref[0] = jnp.exp(-acc)


def _compute_gauss(ea_p, mu_b, inv_b, nl):
    return pl.pallas_call(
        _gauss_body,
        grid=(nl, N_EDGES // EB),
        in_specs=[
            pl.BlockSpec((EB, 8), lambda l, i: (i, 0)),
            pl.BlockSpec((1, 8, 16), lambda l, i: (l, 0, 0)),
            pl.BlockSpec((1, 8, 16), lambda l, i: (l, 0, 0)),
        ],
        out_specs=pl.BlockSpec((1, EB, 16), lambda l, i: (l, i, 0)),
        out_shape=jax.ShapeDtypeStruct((nl, N_EDGES, 16), jnp.float32),
    )(ea_p, mu_b, inv_b)


def _dense_body(h_ref, w_ref, r_ref, b_ref, xh_ref, hr_ref):
    h = h_ref[...]
    xh_ref[...] = jnp.dot(h, w_ref[...], preferred_element_type=jnp.float32)
    hr_ref[...] = (jnp.dot(h, r_ref[...], preferred_element_type=jnp.float32)
                   + b_ref[...])


def _dense(h, w_pad, root_pad, bias_pad, cop):
    n, cip = h.shape
    row = K * cop
    bt = 400
    return pl.pallas_call(
        _dense_body,
        grid=(n // bt,),
        in_specs=[
            pl.BlockSpec((bt, cip), lambda i: (i, 0)),
            pl.BlockSpec((cip, row), lambda i: (0, 0)),
            pl.BlockSpec((cip, cop), lambda i: (0, 0)),
            pl.BlockSpec((1, cop), lambda i: (0, 0)),
        ],
        out_specs=[
            pl.BlockSpec((bt, row), lambda i: (i, 0)),
            pl.BlockSpec((bt, cop), lambda i: (i, 0)),
        ],
        out_shape=[
            jax.ShapeDtypeStruct((n, row), jnp.float32),
            jax.ShapeDtypeStruct((n, cop), jnp.float32),
        ],
    )(h, w_pad, root_pad, bias_pad)


@functools.lru_cache(maxsize=None)
def _make_sc_kernel(cop):
    row = K * cop
    nv = cop // 16
    SUPER = 80 if cop > 96 else 400
    SBC = SUPER // CHUNK
    NSUPER = N_EDGES // NW // SUPER

    def impl(xh, srcr, dstr, gaussr, out, sidx, didx, gbuf, rows0, rows1, msg,
             zbuf, aggsh, sem0, sem1):
        c = lax.axis_index("c")
        s = lax.axis_index("s")
        wid = s * NC + c

        def zrow(i, _):
            for v in range(nv):
                zbuf[i, pl.ds(16 * v, 16)] = jnp.zeros((16,), jnp.float32)
            return 0
        lax.fori_loop(0, ZROWS, zrow, 0)
        rstart = s * ROWS_PER_SUB

        def zcp(j, _):
            pltpu.sync_copy(zbuf, aggsh.at[pl.ds(rstart + j * ZROWS, ZROWS)])
            return 0
        nz = ROWS_PER_SUB // ZROWS + jnp.where(s == NS - 1, 1, 0)
        lax.fori_loop(0, nz, zcp, 0)
        plsc.subcore_barrier()

        ebase_w = wid * (N_EDGES // NW)

        def superchunk(sb, _):
            sbase = ebase_w + sb * SUPER
            pltpu.sync_copy(srcr.at[pl.ds(sbase, SUPER)], sidx)
            pltpu.sync_copy(dstr.at[pl.ds(sbase, SUPER)], didx)
            pltpu.sync_copy(gaussr.at[pl.ds(sbase, SUPER)], gbuf)

            rbufs = (rows0, rows1)
            sems = (sem0, sem1)

            def start(ch, parity):
                svec = sidx[pl.ds(ch * CHUNK, CHUNK)]
                pltpu.async_copy(xh.at[svec], rbufs[parity], sems[parity])

            def wait(ch, parity):
                svec = sidx[pl.ds(ch * CHUNK, CHUNK)]
                pltpu.make_async_copy(xh.at[svec], rbufs[parity],
                                      sems[parity]).wait()

            def process(ch, parity):
                rbuf = rbufs[parity]

                def edge4(i4, _):
                    for u in range(4):
                        e = i4 * 4 + u
                        gv = gbuf[ch * CHUNK + e, pl.ds(0, 16)]
                        g = [gv[k] for k in range(K)]
                        for v in range(nv):
                            acc = g[0] * rbuf[e, pl.ds(16 * v, 16)]
                            for k in range(1, K):
                                acc = acc + g[k] * rbuf[e, pl.ds(k * cop + 16 * v, 16)]
                            msg[e, pl.ds(16 * v, 16)] = acc
                    return 0
                lax.fori_loop(0, CHUNK // 4, edge4, 0)
                dvec = didx[pl.ds(ch * CHUNK, CHUNK)]
                pltpu.sync_copy(msg, aggsh.at[dvec], add=True)

            start(0, 0)
            start(1, 1)

            def pair(j, _):
                c0 = 2 * j
                wait(c0, 0)
                process(c0, 0)

                @pl.when(c0 + 2 < SBC)
                def _():
                    start(c0 + 2, 0)

                @pl.when(c0 + 1 < SBC)
                def _():
                    wait(c0 + 1, 1)
                    process(c0 + 1, 1)

                    @pl.when(c0 + 3 < SBC)
                    def _():
                        start(c0 + 3, 1)
                return 0
            lax.fori_loop(0, (SBC + 1) // 2, pair, 0)
            return 0
        lax.fori_loop(0, NSUPER, superchunk, 0)
        plsc.subcore_barrier()

        @pl.when(s < NS - 1)
        def _():
            pltpu.sync_copy(aggsh.at[pl.ds(rstart, ROWS_PER_SUB)],
                            out.at[c, pl.ds(rstart, ROWS_PER_SUB)])

        @pl.when(s == NS - 1)
        def _():
            last = (NS - 1) * ROWS_PER_SUB
            pltpu.sync_copy(aggsh.at[pl.ds(last, N_NODES - last)],
                            out.at[c, pl.ds(last, N_NODES - last)])

    scratch = [
        pltpu.VMEM((SUPER,), jnp.int32),
        pltpu.VMEM((SUPER,), jnp.int32),
        pltpu.VMEM((SUPER, 16), jnp.float32),
        pltpu.VMEM((CHUNK, row), jnp.float32),
        pltpu.VMEM((CHUNK, row), jnp.float32),
        pltpu.VMEM((CHUNK, cop), jnp.float32),
        pltpu.VMEM((ZROWS, cop), jnp.float32),
        pltpu.VMEM_SHARED((N_NODES, cop), jnp.float32),
        pltpu.SemaphoreType.DMA,
        pltpu.SemaphoreType.DMA,
    ]

    mesh = plsc.VectorSubcoreMesh(core_axis_name="c", subcore_axis_name="s")
    return pl.kernel(
        impl,
        out_type=jax.ShapeDtypeStruct((NC, N_NODES, cop), jnp.float32),
        mesh=mesh,
        compiler_params=pltpu.CompilerParams(use_tc_tiling_on_sc=False),
        scratch_types=scratch,
    )


def _post_body_bn(agg_ref, hr_ref, g_ref, b_ref, out_ref):
    sm = agg_ref[0] + agg_ref[1] + hr_ref[...]
    e = jnp.where(sm > 0, sm, jnp.exp(sm) - 1.0)
    m = jnp.mean(e, axis=0, keepdims=True)
    d = e - m
    v = jnp.mean(d * d, axis=0, keepdims=True)
    out_ref[...] = d * lax.rsqrt(v + 1e-5) * g_ref[...] + b_ref[...]


def _post_body_last(agg_ref, hr_ref, out_ref):
    out_ref[...] = agg_ref[0] + agg_ref[1] + hr_ref[...]


def kernel(x, edge_index, edge_attr, params):
    src = edge_index[0].astype(jnp.int32)
    dst = edge_index[1].astype(jnp.int32)
    nl = len(params)

    ea_p = jnp.zeros((N_EDGES, 8), jnp.float32).at[:, :DIM].set(edge_attr)
    mu_b = jnp.zeros((nl, 8, 16), jnp.float32)
    inv_b = jnp.zeros((nl, 8, 16), jnp.float32)
    for l, p in enumerate(params):
        mu_b = mu_b.at[l, :DIM, :K].set(p["mu"].T)
        inv_b = inv_b.at[l, :DIM, :K].set((0.5 / (p["sigma"] ** 2 + 1e-12)).T)
    gauss_all = _compute_gauss(ea_p, mu_b, inv_b, nl)

    h = x
    for l, p in enumerate(params):
        ci, co = p["root"].shape
        cip = h.shape[1]
        cop = _cop(co)
        w = p["W"].reshape(ci, K, co)
        w_pad = (jnp.zeros((cip, K, cop), jnp.float32)
                 .at[:ci, :, :co].set(w).reshape(cip, K * cop))
        root_pad = jnp.zeros((cip, cop), jnp.float32).at[:ci, :co].set(p["root"])
        bias_pad = jnp.zeros((1, cop), jnp.float32).at[0, :co].set(p["bias"])

        xh, hroot = _dense(h, w_pad, root_pad, bias_pad, cop)
        agg2 = _make_sc_kernel(cop)(xh, src, dst, gauss_all[l])

        if l < nl - 1:
            gamma_pad = jnp.zeros((1, cop), jnp.float32).at[0, :co].set(p["gamma"])
            beta_pad = jnp.zeros((1, cop), jnp.float32).at[0, :co].set(p["beta"])
            h = pl.pallas_call(
                _post_body_bn,
                out_shape=jax.ShapeDtypeStruct((N_NODES, cop), jnp.float32),
            )(agg2, hroot, gamma_pad, beta_pad)
        else:
            h = pl.pallas_call(
                _post_body_last,
                out_shape=jax.ShapeDtypeStruct((N_NODES, cop), jnp.float32),
            )(agg2, hroot)

    return h[:, :params[-1]["root"].shape[1]]
